# baseline (speedup 1.0000x reference)
import functools

import jax
import jax.numpy as jnp
from jax.experimental import pallas as pl
from jax.experimental.pallas import tpu as pltpu

_N = 2_000_000
_C = 17
_MAX_OUT = 5
_IOU_THR = 0.3
_SCORE_THR = 0.5
_IMG_SIZE = 128.0
_CLIP_MAX = 1e8

_NL = 128
_NT = 2_097_152
_NR = _NT // _NL
_BR = 2048
_G = _NR // _BR
_NEG = float("-inf")
_BIG_I = 2 ** 30


def _nms_body(cy_ref, cx_ref, h_ref, w_ref, s_ref, out_ref,
              xs1, ys1, xs2, ys2, sms, smf, smi, sel_i, sel_ok):
    k = pl.program_id(0)
    i = pl.program_id(1)

    rows2d = jax.lax.broadcasted_iota(jnp.int32, (_BR, _NL), 0)
    lanes2d = jax.lax.broadcasted_iota(jnp.int32, (_BR, _NL), 1)

    def update_running(sm):
        bm = jnp.max(sm)
        idxmat = (i * _BR + rows2d) * _NL + lanes2d
        cand = jnp.min(jnp.where(sm == bm, idxmat, _BIG_I))
        cur_m = jnp.where(i == 0, _NEG, smf[0])
        cur_i = jnp.where(i == 0, 0, smi[0])
        better = bm > cur_m
        smf[0] = jnp.where(better, bm, cur_m)
        smi[0] = jnp.where(better, cand, cur_i)

    @pl.when(k == 0)
    def _decode():
        cy = cy_ref[...]
        cx = cx_ref[...]
        h = h_ref[...]
        w = w_ref[...]
        s = s_ref[...]
        y1 = jnp.clip(cy - h * 0.5, 0.0, _CLIP_MAX)
        x1 = jnp.clip(cx - w * 0.5, 0.0, _CLIP_MAX)
        y2 = cy + h * 0.5
        x2 = cx + w * 0.5
        sm = jnp.where(s >= _SCORE_THR, s, _NEG)
        r0 = i * _BR
        xs1[pl.ds(r0, _BR), :] = x1
        ys1[pl.ds(r0, _BR), :] = y1
        xs2[pl.ds(r0, _BR), :] = x2
        ys2[pl.ds(r0, _BR), :] = y2
        sms[pl.ds(r0, _BR), :] = sm
        update_running(sm)

    @pl.when(k > 0)
    def _suppress():
        r0 = i * _BR
        x1 = xs1[pl.ds(r0, _BR), :]
        y1 = ys1[pl.ds(r0, _BR), :]
        x2 = xs2[pl.ds(r0, _BR), :]
        y2 = ys2[pl.ds(r0, _BR), :]
        sm = sms[pl.ds(r0, _BR), :]
        X1 = smf[1]
        Y1 = smf[2]
        X2 = smf[3]
        Y2 = smf[4]
        A = smf[5]
        p_idx = smi[1]
        iw = jnp.maximum(jnp.minimum(x2, X2) - jnp.maximum(x1, X1), 0.0)
        ih = jnp.maximum(jnp.minimum(y2, Y2) - jnp.maximum(y1, Y1), 0.0)
        inter = iw * ih
        areas = (x2 - x1) * (y2 - y1)
        iou = inter / (areas + A - inter + 1e-9)
        idxmat = (i * _BR + rows2d) * _NL + lanes2d
        kill = jnp.logical_or(iou > _IOU_THR, idxmat == p_idx)
        sm = jnp.where(kill, _NEG, sm)
        sms[pl.ds(r0, _BR), :] = sm
        update_running(sm)

    @pl.when(i == _G - 1)
    def _finalize():
        idx = smi[0]
        val = smf[0]
        sel_i[k] = idx
        sel_ok[k] = jnp.where(val > _NEG, 1, 0)
        r = idx // _NL
        l = idx % _NL
        lane1 = jax.lax.broadcasted_iota(jnp.int32, (1, _NL), 1)

        def pick(ref):
            row = ref[pl.ds(r, 1), :]
            return jnp.max(jnp.where(lane1 == l, row, _NEG))

        X1 = pick(xs1)
        Y1 = pick(ys1)
        X2 = pick(xs2)
        Y2 = pick(ys2)
        smf[1] = X1
        smf[2] = Y1
        smf[3] = X2
        smf[4] = Y2
        smf[5] = (X2 - X1) * (Y2 - Y1)
        smi[1] = idx

        @pl.when(k == _MAX_OUT - 1)
        def _emit():
            r8 = jax.lax.broadcasted_iota(jnp.int32, (8, _NL), 0)
            l8 = jax.lax.broadcasted_iota(jnp.int32, (8, _NL), 1)
            acc = jnp.zeros((8, _NL), jnp.int32)
            for j in range(_MAX_OUT):
                acc = jnp.where((r8 == 0) & (l8 == j), sel_i[j], acc)
                acc = jnp.where((r8 == 1) & (l8 == j), sel_ok[j], acc)
            out_ref[...] = acc


_NF = (_N * _C) // _NL
_GA = _NF // 8


def _gather_body(sref, detA_ref, detB_ref, out_ref):
    j = pl.program_id(0)
    blkA = detA_ref[...]
    blkB = detB_ref[...]
    idx = sref[j]
    a = (idx * _C) // (_NL * 8)
    off = idx * _C - a * _NL * 8
    ok = sref[_MAX_OUT + j] > 0
    flatpos = (jax.lax.broadcasted_iota(jnp.int32, (8, _NL), 0) * _NL
               + jax.lax.broadcasted_iota(jnp.int32, (8, _NL), 1))
    vals = []
    for c in range(_C):
        t = off + c
        va = jnp.sum(jnp.where(flatpos == t, blkA, 0.0))
        vb = jnp.sum(jnp.where(flatpos + _NL * 8 == t, blkB, 0.0))
        vals.append(va + vb)
    row = jnp.stack(vals).reshape(1, 1, _C)
    lanes = jax.lax.broadcasted_iota(jnp.int32, (1, 1, _C), 2)
    scaled = jnp.where(lanes < _C - 1, row * _IMG_SIZE, row)
    out_ref[...] = jnp.where(ok, scaled, 0.0)


@jax.jit
def kernel(detections):
    det = detections.reshape(_N, _C)
    pad = jnp.zeros((_NT - _N,), jnp.float32)

    def col(kk):
        return jnp.concatenate([det[:, kk], pad]).reshape(_NR, _NL)

    cy, cx, hh, ww, sc = col(0), col(1), col(2), col(3), col(_C - 1)

    in_spec = pl.BlockSpec(
        (_BR, _NL), lambda k, i: (jnp.where(k == 0, i, 0), 0))
    sel = pl.pallas_call(
        _nms_body,
        out_shape=jax.ShapeDtypeStruct((8, _NL), jnp.int32),
        grid=(_MAX_OUT, _G),
        in_specs=[in_spec] * 5,
        out_specs=pl.BlockSpec((8, _NL), lambda k, i: (0, 0)),
        scratch_shapes=[
            pltpu.VMEM((_NR, _NL), jnp.float32),
            pltpu.VMEM((_NR, _NL), jnp.float32),
            pltpu.VMEM((_NR, _NL), jnp.float32),
            pltpu.VMEM((_NR, _NL), jnp.float32),
            pltpu.VMEM((_NR, _NL), jnp.float32),
            pltpu.SMEM((8,), jnp.float32),
            pltpu.SMEM((8,), jnp.int32),
            pltpu.SMEM((8,), jnp.int32),
            pltpu.SMEM((8,), jnp.int32),
        ],
        compiler_params=pltpu.CompilerParams(
            dimension_semantics=("arbitrary", "arbitrary"),
            vmem_limit_bytes=64 * 1024 * 1024,
        ),
        name="nms_core",
    )(cy, cx, hh, ww, sc)

    idxs = jnp.minimum(sel[0, :_MAX_OUT], _N - 1)
    oks = sel[1, :_MAX_OUT]
    scal = jnp.concatenate([idxs, oks])
    det_flat = detections.reshape(_NF, _NL)

    out3 = pl.pallas_call(
        _gather_body,
        out_shape=jax.ShapeDtypeStruct((_MAX_OUT, 1, _C), jnp.float32),
        grid_spec=pltpu.PrefetchScalarGridSpec(
            num_scalar_prefetch=1,
            grid=(_MAX_OUT,),
            in_specs=[
                pl.BlockSpec(
                    (8, _NL),
                    lambda j, sref: ((sref[j] * _C) // (_NL * 8), 0)),
                pl.BlockSpec(
                    (8, _NL),
                    lambda j, sref: (jnp.minimum(
                        (sref[j] * _C) // (_NL * 8) + 1, _GA), 0)),
            ],
            out_specs=pl.BlockSpec((1, 1, _C), lambda j, sref: (j, 0, 0)),
        ),
        name="nms_gather",
    )(scal, det_flat, det_flat)

    return out3.reshape(_MAX_OUT, _C)


# speedup vs baseline: 1.6272x; 1.6272x over previous
import jax
import jax.numpy as jnp
from jax.experimental import pallas as pl
from jax.experimental.pallas import tpu as pltpu

_N = 2_000_000
_C = 17
_MAX_OUT = 5
_IOU_THR = 0.3
_SCORE_THR = 0.5
_IMG_SIZE = 128.0
_CLIP_MAX = 1e8

_NL = 128
_BRD = 16_000
_CHK = _BRD // _NL
_G = _N // _BRD
_NEG = float("-inf")
_BIG_I = 2 ** 30


def _nms_body(det_ref, out_ref, xs1, ys1, xs2, ys2, sms, smf, smi,
              sel_i, sel_ok):
    g = pl.program_id(0)

    def update_running(sm, idxmat, first):
        bm = jnp.max(sm)
        cand = jnp.min(jnp.where(sm == bm, idxmat, _BIG_I))
        cur_m = jnp.where(first, _NEG, smf[0])
        cur_i = jnp.where(first, 0, smi[0])
        better = bm > cur_m
        smf[0] = jnp.where(better, bm, cur_m)
        smi[0] = jnp.where(better, cand, cur_i)

    @pl.when(g < _G)
    def _decode():
        d = det_ref[...].reshape(_CHK, _NL, _C)
        t = jnp.swapaxes(d, 1, 2)
        cy = t[:, 0, :]
        cx = t[:, 1, :]
        h = t[:, 2, :]
        w = t[:, 3, :]
        s = t[:, _C - 1, :]
        y1 = jnp.clip(cy - h * 0.5, 0.0, _CLIP_MAX)
        x1 = jnp.clip(cx - w * 0.5, 0.0, _CLIP_MAX)
        y2 = cy + h * 0.5
        x2 = cx + w * 0.5
        sm = jnp.where(s >= _SCORE_THR, s, _NEG)
        xs1[g] = x1
        ys1[g] = y1
        xs2[g] = x2
        ys2[g] = y2
        sms[g] = sm
        rows2 = jax.lax.broadcasted_iota(jnp.int32, (_CHK, _NL), 0)
        lanes2 = jax.lax.broadcasted_iota(jnp.int32, (_CHK, _NL), 1)
        idxmat = (g * _CHK + rows2) * _NL + lanes2
        update_running(sm, idxmat, g == 0)

    @pl.when(g >= _G)
    def _suppress():
        X1 = smf[1]
        Y1 = smf[2]
        X2 = smf[3]
        Y2 = smf[4]
        A = smf[5]
        p_idx = smi[1]
        smf[0] = _NEG
        smi[0] = 0
        rows2 = jax.lax.broadcasted_iota(jnp.int32, (_CHK, _NL), 0)
        lanes2 = jax.lax.broadcasted_iota(jnp.int32, (_CHK, _NL), 1)

        def body(p, carry):
            x1 = xs1[p]
            y1 = ys1[p]
            x2 = xs2[p]
            y2 = ys2[p]
            sm = sms[p]
            iw = jnp.maximum(jnp.minimum(x2, X2) - jnp.maximum(x1, X1), 0.0)
            ih = jnp.maximum(jnp.minimum(y2, Y2) - jnp.maximum(y1, Y1), 0.0)
            inter = iw * ih
            areas = (x2 - x1) * (y2 - y1)
            iou = inter / (areas + A - inter + 1e-9)
            idxmat = (p * _CHK + rows2) * _NL + lanes2
            kill = jnp.logical_or(iou > _IOU_THR, idxmat == p_idx)
            sm = jnp.where(kill, _NEG, sm)
            sms[p] = sm
            bm = jnp.max(sm)
            cand = jnp.min(jnp.where(sm == bm, idxmat, _BIG_I))
            cur_m = smf[0]
            cur_i = smi[0]
            better = bm > cur_m
            smf[0] = jnp.where(better, bm, cur_m)
            smi[0] = jnp.where(better, cand, cur_i)
            return carry

        jax.lax.fori_loop(0, _G, body, 0)

    @pl.when(jnp.logical_or(g == _G - 1, g >= _G))
    def _finalize():
        idx = smi[0]
        val = smf[0]
        k = jnp.maximum(g - (_G - 1), 0)
        sel_i[k] = idx
        sel_ok[k] = jnp.where(val > _NEG, 1, 0)
        gg = idx // _BRD
        cc = (idx // _NL) % _CHK
        ll = idx % _NL
        lane1 = jax.lax.broadcasted_iota(jnp.int32, (1, _NL), 1)

        def pick(ref):
            row = ref[gg, pl.ds(cc, 1), :]
            return jnp.max(jnp.where(lane1 == ll, row, _NEG))

        X1 = pick(xs1)
        Y1 = pick(ys1)
        X2 = pick(xs2)
        Y2 = pick(ys2)
        smf[1] = X1
        smf[2] = Y1
        smf[3] = X2
        smf[4] = Y2
        smf[5] = (X2 - X1) * (Y2 - Y1)
        smi[1] = idx

        @pl.when(g == _G + _MAX_OUT - 2)
        def _emit():
            r8 = jax.lax.broadcasted_iota(jnp.int32, (8, _NL), 0)
            l8 = jax.lax.broadcasted_iota(jnp.int32, (8, _NL), 1)
            acc = jnp.zeros((8, _NL), jnp.int32)
            for j in range(_MAX_OUT):
                acc = jnp.where((r8 == 0) & (l8 == j), sel_i[j], acc)
                acc = jnp.where((r8 == 1) & (l8 == j), sel_ok[j], acc)
            out_ref[...] = acc


def _gather_body(sref, det_ref, out_ref):
    j = pl.program_id(0)
    blk = det_ref[...]
    sub = sref[j] % 8
    ok = sref[_MAX_OUT + j] > 0
    subs = jax.lax.broadcasted_iota(jnp.int32, (1, 8, _C), 1)
    row = jnp.max(jnp.where(subs == sub, blk, _NEG), axis=1, keepdims=True)
    lanes = jax.lax.broadcasted_iota(jnp.int32, (1, 1, _C), 2)
    scaled = jnp.where(lanes < _C - 1, row * _IMG_SIZE, row)
    out_ref[...] = jnp.where(ok, scaled, 0.0)


@jax.jit
def kernel(detections):
    sel = pl.pallas_call(
        _nms_body,
        out_shape=jax.ShapeDtypeStruct((8, _NL), jnp.int32),
        grid=(_G + _MAX_OUT - 1,),
        in_specs=[pl.BlockSpec((1, _BRD, _C),
                               lambda g: (0, jnp.minimum(g, _G - 1), 0))],
        out_specs=pl.BlockSpec((8, _NL), lambda g: (0, 0)),
        scratch_shapes=[
            pltpu.VMEM((_G, _CHK, _NL), jnp.float32),
            pltpu.VMEM((_G, _CHK, _NL), jnp.float32),
            pltpu.VMEM((_G, _CHK, _NL), jnp.float32),
            pltpu.VMEM((_G, _CHK, _NL), jnp.float32),
            pltpu.VMEM((_G, _CHK, _NL), jnp.float32),
            pltpu.SMEM((8,), jnp.float32),
            pltpu.SMEM((8,), jnp.int32),
            pltpu.SMEM((8,), jnp.int32),
            pltpu.SMEM((8,), jnp.int32),
        ],
        compiler_params=pltpu.CompilerParams(
            dimension_semantics=("arbitrary",),
            vmem_limit_bytes=64 * 1024 * 1024,
        ),
        name="nms_core",
    )(detections)

    idxs = jnp.minimum(sel[0, :_MAX_OUT], _N - 1)
    oks = sel[1, :_MAX_OUT]
    scal = jnp.concatenate([idxs, oks])

    out3 = pl.pallas_call(
        _gather_body,
        out_shape=jax.ShapeDtypeStruct((_MAX_OUT, 1, _C), jnp.float32),
        grid_spec=pltpu.PrefetchScalarGridSpec(
            num_scalar_prefetch=1,
            grid=(_MAX_OUT,),
            in_specs=[pl.BlockSpec((1, 8, _C),
                                   lambda j, sref: (0, sref[j] // 8, 0))],
            out_specs=pl.BlockSpec((1, 1, _C), lambda j, sref: (j, 0, 0)),
        ),
        name="nms_gather",
    )(scal, detections)

    return out3.reshape(_MAX_OUT, _C)


# speedup vs baseline: 1.6449x; 1.0108x over previous
import jax
import jax.numpy as jnp
from jax.experimental import pallas as pl
from jax.experimental.pallas import tpu as pltpu

_N = 2_000_000
_C = 17
_MAX_OUT = 5
_IOU_THR = 0.3
_SCORE_THR = 0.5
_IMG_SIZE = 128.0
_CLIP_MAX = 1e8

_NL = 128
_BRD = 16_000
_NQ = 5
_QW = _BRD // _NQ
_CHK = _BRD // _NL
_CHQ = _QW // _NL
_G = _N // _BRD
_NEG = float("-inf")
_BIG_I = 2 ** 30


def _nms_body(det_ref0, det_ref1, det_ref2, det_ref3, det_ref4,
              out_ref, xs1, ys1, xs2, ys2, sms, smf, smi,
              sel_i, sel_ok):
    g = pl.program_id(0)
    det_refs = (det_ref0, det_ref1, det_ref2, det_ref3, det_ref4)

    def update_running(sm, idxmat, first):
        bm = jnp.max(sm)
        cand = jnp.min(jnp.where(sm == bm, idxmat, _BIG_I))
        cur_m = jnp.where(first, _NEG, smf[0])
        cur_i = jnp.where(first, 0, smi[0])
        better = bm > cur_m
        smf[0] = jnp.where(better, bm, cur_m)
        smi[0] = jnp.where(better, cand, cur_i)

    @pl.when(g < _G)
    def _decode():
        parts = []
        for r in det_refs:
            dq = r[...].reshape(_CHQ, _NL, _C)
            parts.append(jnp.swapaxes(dq, 1, 2))
        t = jnp.concatenate(parts, axis=0)
        cy = t[:, 0, :]
        cx = t[:, 1, :]
        h = t[:, 2, :]
        w = t[:, 3, :]
        s = t[:, _C - 1, :]
        y1 = jnp.clip(cy - h * 0.5, 0.0, _CLIP_MAX)
        x1 = jnp.clip(cx - w * 0.5, 0.0, _CLIP_MAX)
        y2 = cy + h * 0.5
        x2 = cx + w * 0.5
        sm = jnp.where(s >= _SCORE_THR, s, _NEG)
        xs1[g] = x1
        ys1[g] = y1
        xs2[g] = x2
        ys2[g] = y2
        sms[g] = sm
        rows2 = jax.lax.broadcasted_iota(jnp.int32, (_CHK, _NL), 0)
        lanes2 = jax.lax.broadcasted_iota(jnp.int32, (_CHK, _NL), 1)
        idxmat = (g * _CHK + rows2) * _NL + lanes2
        update_running(sm, idxmat, g == 0)

    @pl.when(g >= _G)
    def _suppress():
        X1 = smf[1]
        Y1 = smf[2]
        X2 = smf[3]
        Y2 = smf[4]
        A = smf[5]
        p_idx = smi[1]
        smf[0] = _NEG
        smi[0] = 0
        rows2 = jax.lax.broadcasted_iota(jnp.int32, (_CHK, _NL), 0)
        lanes2 = jax.lax.broadcasted_iota(jnp.int32, (_CHK, _NL), 1)

        def body(p, carry):
            x1 = xs1[p]
            y1 = ys1[p]
            x2 = xs2[p]
            y2 = ys2[p]
            sm = sms[p]
            iw = jnp.maximum(jnp.minimum(x2, X2) - jnp.maximum(x1, X1), 0.0)
            ih = jnp.maximum(jnp.minimum(y2, Y2) - jnp.maximum(y1, Y1), 0.0)
            inter = iw * ih
            areas = (x2 - x1) * (y2 - y1)
            iou = inter / (areas + A - inter + 1e-9)
            idxmat = (p * _CHK + rows2) * _NL + lanes2
            kill = jnp.logical_or(iou > _IOU_THR, idxmat == p_idx)
            sm = jnp.where(kill, _NEG, sm)
            sms[p] = sm
            bm = jnp.max(sm)
            cand = jnp.min(jnp.where(sm == bm, idxmat, _BIG_I))
            cur_m = smf[0]
            cur_i = smi[0]
            better = bm > cur_m
            smf[0] = jnp.where(better, bm, cur_m)
            smi[0] = jnp.where(better, cand, cur_i)
            return carry

        jax.lax.fori_loop(0, _G, body, 0)

    @pl.when(jnp.logical_or(g == _G - 1, g >= _G))
    def _finalize():
        idx = smi[0]
        val = smf[0]
        k = jnp.maximum(g - (_G - 1), 0)
        sel_i[k] = idx
        sel_ok[k] = jnp.where(val > _NEG, 1, 0)
        gg = idx // _BRD
        cc = (idx // _NL) % _CHK
        ll = idx % _NL
        lane1 = jax.lax.broadcasted_iota(jnp.int32, (1, _NL), 1)

        def pick(ref):
            row = ref[gg, pl.ds(cc, 1), :]
            return jnp.max(jnp.where(lane1 == ll, row, _NEG))

        X1 = pick(xs1)
        Y1 = pick(ys1)
        X2 = pick(xs2)
        Y2 = pick(ys2)
        smf[1] = X1
        smf[2] = Y1
        smf[3] = X2
        smf[4] = Y2
        smf[5] = (X2 - X1) * (Y2 - Y1)
        smi[1] = idx

        @pl.when(g == _G + _MAX_OUT - 2)
        def _emit():
            r8 = jax.lax.broadcasted_iota(jnp.int32, (8, _NL), 0)
            l8 = jax.lax.broadcasted_iota(jnp.int32, (8, _NL), 1)
            acc = jnp.zeros((8, _NL), jnp.int32)
            for j in range(_MAX_OUT):
                acc = jnp.where((r8 == 0) & (l8 == j), sel_i[j], acc)
                acc = jnp.where((r8 == 1) & (l8 == j), sel_ok[j], acc)
            out_ref[...] = acc


def _gather_body(sref, det_ref, out_ref):
    j = pl.program_id(0)
    blk = det_ref[...]
    sub = sref[j] % 8
    ok = sref[_MAX_OUT + j] > 0
    subs = jax.lax.broadcasted_iota(jnp.int32, (1, 8, _C), 1)
    row = jnp.max(jnp.where(subs == sub, blk, _NEG), axis=1, keepdims=True)
    lanes = jax.lax.broadcasted_iota(jnp.int32, (1, 1, _C), 2)
    scaled = jnp.where(lanes < _C - 1, row * _IMG_SIZE, row)
    out_ref[...] = jnp.where(ok, scaled, 0.0)


@jax.jit
def kernel(detections):
    sel = pl.pallas_call(
        _nms_body,
        out_shape=jax.ShapeDtypeStruct((8, _NL), jnp.int32),
        grid=(_G + _MAX_OUT - 1,),
        in_specs=[
            pl.BlockSpec(
                (1, _QW, _C),
                (lambda q: (lambda g: (0, jnp.minimum(g, _G - 1) * _NQ + q,
                                       0)))(q))
            for q in range(_NQ)
        ],
        out_specs=pl.BlockSpec((8, _NL), lambda g: (0, 0)),
        scratch_shapes=[
            pltpu.VMEM((_G, _CHK, _NL), jnp.float32),
            pltpu.VMEM((_G, _CHK, _NL), jnp.float32),
            pltpu.VMEM((_G, _CHK, _NL), jnp.float32),
            pltpu.VMEM((_G, _CHK, _NL), jnp.float32),
            pltpu.VMEM((_G, _CHK, _NL), jnp.float32),
            pltpu.SMEM((8,), jnp.float32),
            pltpu.SMEM((8,), jnp.int32),
            pltpu.SMEM((8,), jnp.int32),
            pltpu.SMEM((8,), jnp.int32),
        ],
        compiler_params=pltpu.CompilerParams(
            dimension_semantics=("arbitrary",),
            vmem_limit_bytes=64 * 1024 * 1024,
        ),
        name="nms_core",
    )(detections, detections, detections, detections, detections)

    idxs = jnp.minimum(sel[0, :_MAX_OUT], _N - 1)
    oks = sel[1, :_MAX_OUT]
    scal = jnp.concatenate([idxs, oks])

    out3 = pl.pallas_call(
        _gather_body,
        out_shape=jax.ShapeDtypeStruct((_MAX_OUT, 1, _C), jnp.float32),
        grid_spec=pltpu.PrefetchScalarGridSpec(
            num_scalar_prefetch=1,
            grid=(_MAX_OUT,),
            in_specs=[pl.BlockSpec((1, 8, _C),
                                   lambda j, sref: (0, sref[j] // 8, 0))],
            out_specs=pl.BlockSpec((1, 1, _C), lambda j, sref: (j, 0, 0)),
        ),
        name="nms_gather",
    )(scal, detections)

    return out3.reshape(_MAX_OUT, _C)
